# baseline (speedup 1.0000x reference)
import functools

import jax
import jax.numpy as jnp
from jax import lax
from jax.experimental import pallas as pl
from jax.experimental.pallas import tpu as pltpu
from jax.experimental.pallas import tpu_sc as plsc

BATCH = 16384
DIM = 64
NC = 2
NS = 16
NW = NC * NS
ROWS_PER_W = BATCH // NW
CHUNK = 128
NCHUNK = ROWS_PER_W // CHUNK
GPC = CHUNK // 16


def _sc_body(head_hbm, tail_hbm, rel_hbm, ent_hbm, relemb_hbm, out_hbm,
             hidx, tidx, ridx, gidx, hbuf, tbuf, rbuf, oscr, sem):
    c = lax.axis_index("c")
    s = lax.axis_index("s")
    wid = s * NC + c
    base = wid * ROWS_PER_W

    pltpu.sync_copy(head_hbm.at[pl.ds(base, ROWS_PER_W)], hidx)
    pltpu.sync_copy(tail_hbm.at[pl.ds(base, ROWS_PER_W)], tidx)
    pltpu.sync_copy(rel_hbm.at[pl.ds(base, ROWS_PER_W)], ridx)

    iota16 = lax.iota(jnp.int32, 16)

    def chunk_body(ck, carry):
        off = pl.multiple_of(ck * CHUNK, CHUNK)
        for v in range(CHUNK // 16):
            gidx[pl.ds(16 * v, 16)] = (
                lax.shift_right_logical(hidx[pl.ds(off + 16 * v, 16)], 1))
        cph = pltpu.async_copy(ent_hbm.at[gidx], hbuf, sem)
        cph.wait()
        for v in range(CHUNK // 16):
            gidx[pl.ds(16 * v, 16)] = (
                lax.shift_right_logical(tidx[pl.ds(off + 16 * v, 16)], 1))
        cpt = pltpu.async_copy(ent_hbm.at[gidx], tbuf, sem)
        cpt.wait()
        for v in range(CHUNK // 16):
            gidx[pl.ds(16 * v, 16)] = (
                lax.shift_right_logical(ridx[pl.ds(off + 16 * v, 16)], 1))
        cpr = pltpu.async_copy(relemb_hbm.at[gidx], rbuf, sem)
        cpr.wait()

        for g in range(GPC):
            goff = off + g * 16
            slot = g * 16 + iota16
            hsel = (hidx[pl.ds(goff, 16)] & 1) * 64
            tsel = (tidx[pl.ds(goff, 16)] & 1) * 64
            rsel = (ridx[pl.ds(goff, 16)] & 1) * 64
            acc = jnp.zeros((16,), jnp.float32)
            for d in range(DIM):
                dv = jnp.full((16,), d, jnp.int32)
                h = plsc.load_gather(hbuf, [slot, hsel + dv])
                t = plsc.load_gather(tbuf, [slot, tsel + dv])
                r = plsc.load_gather(rbuf, [slot, rsel + dv])
                acc = acc + h * t * r
            score = 1.0 / (1.0 + jnp.exp(-acc))
            oscr[pl.ds(goff, 16)] = score
        return carry

    lax.fori_loop(0, NCHUNK, chunk_body, 0)

    pltpu.sync_copy(oscr, out_hbm.at[pl.ds(base, ROWS_PER_W)])


@functools.partial(
    pl.kernel,
    mesh=plsc.VectorSubcoreMesh(core_axis_name="c", subcore_axis_name="s"),
    out_type=jax.ShapeDtypeStruct((BATCH,), jnp.float32),
    compiler_params=pltpu.CompilerParams(needs_layout_passes=False),
    scratch_types=[
        pltpu.VMEM((ROWS_PER_W,), jnp.int32),
        pltpu.VMEM((ROWS_PER_W,), jnp.int32),
        pltpu.VMEM((ROWS_PER_W,), jnp.int32),
        pltpu.VMEM((CHUNK,), jnp.int32),
        pltpu.VMEM((CHUNK, 2 * DIM), jnp.float32),
        pltpu.VMEM((CHUNK, 2 * DIM), jnp.float32),
        pltpu.VMEM((CHUNK, 2 * DIM), jnp.float32),
        pltpu.VMEM((ROWS_PER_W,), jnp.float32),
        pltpu.SemaphoreType.DMA,
    ],
)
def _distmult_sc(*args):
    _sc_body(*args)


def kernel(head, tail, relation, entity_embed, relation_embed):
    ent2 = entity_embed.reshape(entity_embed.shape[0] // 2, 2 * DIM)
    rel2 = relation_embed.reshape(relation_embed.shape[0] // 2, 2 * DIM)
    return _distmult_sc(head.astype(jnp.int32), tail.astype(jnp.int32),
                        relation.astype(jnp.int32), ent2, rel2)
